# baseline (speedup 1.0000x reference)
import functools

import jax
import jax.numpy as jnp
from jax import lax
from jax.experimental import pallas as pl
from jax.experimental.pallas import tpu as pltpu
from jax.experimental.pallas import tpu_sc as plsc

N = 1000000
D = 64
B = 16384

NC = 2
NS = 16
NW = NC * NS
BPW = B // NW
L = 16
CHUNK = 256
GW = 128


def _bpr_sc(users, pos, neg, E2, pos_out, neg_out,
            idx_us, idx_ps, idx_ns, pr_us, pr_ps, pr_ns,
            u_rows, p_rows, n_rows, o_p, o_n, sem):
    wid = lax.axis_index("s") * NC + lax.axis_index("c")
    base = wid * BPW

    pltpu.sync_copy(users.at[pl.ds(base, BPW)], idx_us)
    pltpu.sync_copy(pos.at[pl.ds(base, BPW)], idx_ps)
    pltpu.sync_copy(neg.at[pl.ds(base, BPW)], idx_ns)

    def halve_body(v, _):
        s = pl.ds(v * L, L)
        pr_us[s] = lax.shift_right_logical(idx_us[s], 1)
        pr_ps[s] = lax.shift_right_logical(idx_ps[s], 1)
        pr_ns[s] = lax.shift_right_logical(idx_ns[s], 1)
        return 0

    lax.fori_loop(0, BPW // L, halve_body, 0)

    lane = lax.iota(jnp.int32, L)

    def chunk_body(c, _):
        coff = c * CHUNK

        for j in range(CHUNK // GW):
            s = pl.ds(coff + j * GW, GW)
            d = pl.ds(j * GW, GW)
            pltpu.async_copy(E2.at[pr_us.at[s]], u_rows.at[d], sem)
            pltpu.async_copy(E2.at[pr_ps.at[s]], p_rows.at[d], sem)
            pltpu.async_copy(E2.at[pr_ns.at[s]], n_rows.at[d], sem)

        pltpu.make_async_copy(E2.at[pl.ds(0, CHUNK)], u_rows, sem).wait()
        pltpu.make_async_copy(E2.at[pl.ds(0, CHUNK)], p_rows, sem).wait()
        pltpu.make_async_copy(E2.at[pl.ds(0, CHUNK)], n_rows, sem).wait()

        def group_body(g, _):
            s = pl.ds(coff + g * L, L)
            rows = g * L + lane
            cu = (idx_us[s] & 1) * D
            cp = (idx_ps[s] & 1) * D
            cn = (idx_ns[s] & 1) * D

            def d_body(d, accs):
                acc_p, acc_n = accs
                u = plsc.load_gather(u_rows, [rows, cu + d])
                p = plsc.load_gather(p_rows, [rows, cp + d])
                n = plsc.load_gather(n_rows, [rows, cn + d])
                return (acc_p + u * p, acc_n + u * n)

            acc_p, acc_n = lax.fori_loop(
                0, D, d_body,
                (jnp.zeros((L,), jnp.float32), jnp.zeros((L,), jnp.float32)))
            o_p[s] = acc_p
            o_n[s] = acc_n
            return 0

        lax.fori_loop(0, CHUNK // L, group_body, 0)
        return 0

    lax.fori_loop(0, BPW // CHUNK, chunk_body, 0)

    pltpu.sync_copy(o_p, pos_out.at[pl.ds(base, BPW)])
    pltpu.sync_copy(o_n, neg_out.at[pl.ds(base, BPW)])


@functools.cache
def _build():
    mesh = plsc.VectorSubcoreMesh(core_axis_name="c", subcore_axis_name="s",
                                  num_cores=NC, num_subcores=NS)
    return pl.kernel(
        _bpr_sc,
        out_type=(
            jax.ShapeDtypeStruct((B,), jnp.float32),
            jax.ShapeDtypeStruct((B,), jnp.float32),
        ),
        mesh=mesh,
        scratch_types=[
            pltpu.VMEM((BPW,), jnp.int32),
            pltpu.VMEM((BPW,), jnp.int32),
            pltpu.VMEM((BPW,), jnp.int32),
            pltpu.VMEM((BPW,), jnp.int32),
            pltpu.VMEM((BPW,), jnp.int32),
            pltpu.VMEM((BPW,), jnp.int32),
            pltpu.VMEM((CHUNK, 2 * D), jnp.float32),
            pltpu.VMEM((CHUNK, 2 * D), jnp.float32),
            pltpu.VMEM((CHUNK, 2 * D), jnp.float32),
            pltpu.VMEM((BPW,), jnp.float32),
            pltpu.VMEM((BPW,), jnp.float32),
            pltpu.SemaphoreType.DMA,
        ],
        compiler_params=pltpu.CompilerParams(needs_layout_passes=False,
                                             use_tc_tiling_on_sc=True),
    )


def kernel(users, pos, neg, E):
    E2 = jnp.reshape(E, (N // 2, 2 * D))
    return _build()(users.astype(jnp.int32), pos.astype(jnp.int32),
                    neg.astype(jnp.int32), E2)


# speedup vs baseline: 1.8174x; 1.8174x over previous
import functools

import jax
import jax.numpy as jnp
from jax import lax
from jax.experimental import pallas as pl
from jax.experimental.pallas import tpu as pltpu
from jax.experimental.pallas import tpu_sc as plsc

N = 1000000
D = 64
B = 16384

NC = 2
NS = 16
NW = NC * NS
BPW = B // NW
L = 16
NBLK = (N + 127) // 128
HMAX = 4096
RB = 128
CHUNK = 256


def _splat(ref, i):
    return plsc.load_gather(ref, [jnp.full((L,), i, jnp.int32)])


def _sweep_sc(users, pos, neg, Et, staging,
              iu, ip, inn, hraw_i, hraw_t, hsrt_i, hsrt_t,
              cnt, starts, basec, blk0, blk1, rowbuf, dummy, sem_in, sem_out):
    wid = lax.axis_index("s") * NC + lax.axis_index("c")
    b_lo = wid * NBLK // NW
    b_hi = (wid + 1) * NBLK // NW
    lo = b_lo * 128
    hi = b_hi * 128

    pltpu.sync_copy(users, iu)
    pltpu.sync_copy(pos, ip)
    pltpu.sync_copy(neg, inn)

    lane = lax.iota(jnp.int32, L)

    def scan_t(ref, t, off0):
        def scan_g(g, off):
            v = ref[pl.ds(g * L, L)]
            m = (v >= lo) & (v < hi)
            plsc.store_compressed(hraw_i.at[pl.ds(off, L)], v, mask=m)
            tag = t * B + g * L + lane
            plsc.store_compressed(hraw_t.at[pl.ds(off, L)], tag, mask=m)
            return off + plsc.all_reduce_population_count(m)[0]
        return lax.fori_loop(0, B // L, scan_g, off0)

    nhits = scan_t(iu, 0, jnp.int32(0))
    nhits = scan_t(ip, 1, nhits)
    nhits = scan_t(inn, 2, nhits)

    zeros = jnp.zeros((L,), jnp.int32)
    def zero_b(k, _):
        cnt[pl.ds(k * L, L)] = zeros
        return 0
    lax.fori_loop(0, 256 // L, zero_b, 0)

    one0 = jnp.where(lane == 0, 1, 0)
    m0 = lane == 0

    def count_b(h, _):
        v = _splat(hraw_i, h)
        blk = lax.shift_right_logical(v, 7) - b_lo
        plsc.addupdate_scatter(cnt, [blk & 255], one0, mask=m0)
        return 0
    lax.fori_loop(0, nhits, count_b, 0)

    def cum_b(k, carry):
        v = cnt[pl.ds(k * L, L)]
        inc = plsc.cumsum(v) + carry
        exc = inc - v
        starts[pl.ds(k * L, L)] = exc
        basec[pl.ds(k * L, L)] = exc
        return inc[L - 1]
    lax.fori_loop(0, 256 // L, cum_b, jnp.int32(0))

    def place_b(h, _):
        v = _splat(hraw_i, h)
        tg = _splat(hraw_t, h)
        blk = (lax.shift_right_logical(v, 7) - b_lo) & 255
        p = plsc.load_gather(basec, [blk]) & (HMAX - 1)
        plsc.store_scatter(hsrt_i, [p], v, mask=m0)
        plsc.store_scatter(hsrt_t, [p], tg, mask=m0)
        plsc.addupdate_scatter(basec, [blk], one0, mask=m0)
        return 0
    lax.fori_loop(0, nhits, place_b, 0)

    nb = b_hi - b_lo
    bufs = (blk0, blk1)

    pltpu.async_copy(Et.at[:, pl.ds(b_lo * 128, 128)], blk0, sem_in)

    d4 = [jnp.full((L,), c * L, jnp.int32) + lane for c in range(D // L)]

    def sweep_b(b, _):
        for par in range(2):
            @pl.when(b % 2 == par)
            def _():
                pltpu.make_async_copy(
                    Et.at[:, pl.ds(0, 128)], bufs[par], sem_in).wait()

        @pl.when(b + 1 < nb)
        def _():
            for par in range(2):
                @pl.when((b + 1) % 2 == par)
                def _():
                    pltpu.async_copy(
                        Et.at[:, pl.ds((b_lo + b + 1) * 128, 128)],
                        bufs[par], sem_in)

        h0 = _splat(starts, b)[0]
        h1 = _splat(starts, b + 1)[0]

        def hit_body(h, _):
            v = _splat(hsrt_i, h)
            tg = _splat(hsrt_t, h)[0]
            il = v & 127
            slot = h & (RB - 1)
            for par in range(2):
                @pl.when(b % 2 == par)
                def _():
                    for c in range(D // L):
                        rowbuf[slot, pl.ds(c * L, L)] = plsc.load_gather(
                            bufs[par], [d4[c], il])
            pltpu.async_copy(rowbuf.at[pl.ds(slot, 1)],
                             staging.at[pl.ds(tg, 1)], sem_out)
            return 0

        lax.fori_loop(h0, h1, hit_body, 0)
        return 0

    lax.fori_loop(0, nb, sweep_b, 0)

    def drain_b(k, _):
        pltpu.make_async_copy(users.at[pl.ds(0, 2 * D)], dummy, sem_out).wait()
        return 0
    lax.fori_loop(0, nhits, drain_b, 0)


def _dots_tc(u_ref, p_ref, n_ref, po_ref, no_ref):
    u = u_ref[...]
    po_ref[...] = jnp.sum(u * p_ref[...], axis=1)
    no_ref[...] = jnp.sum(u * n_ref[...], axis=1)


_TCB = 2048


def _dots_sc(staging, pos_out, neg_out, u_buf, p_buf, n_buf, o_p, o_n, sem):
    wid = lax.axis_index("s") * NC + lax.axis_index("c")
    base = wid * BPW
    lane = lax.iota(jnp.int32, L)

    def chunk_body(ch, _):
        coff = ch * CHUNK

        def row_body(i, _):
            dst = pl.ds(i, 1)
            pltpu.async_copy(staging.at[pl.ds(base + coff + i, 1)],
                             u_buf.at[dst], sem)
            pltpu.async_copy(staging.at[pl.ds(B + base + coff + i, 1)],
                             p_buf.at[dst], sem)
            pltpu.async_copy(staging.at[pl.ds(2 * B + base + coff + i, 1)],
                             n_buf.at[dst], sem)
            return 0

        lax.fori_loop(0, CHUNK, row_body, 0)
        pltpu.make_async_copy(staging.at[pl.ds(0, CHUNK)], u_buf, sem).wait()
        pltpu.make_async_copy(staging.at[pl.ds(0, CHUNK)], p_buf, sem).wait()
        pltpu.make_async_copy(staging.at[pl.ds(0, CHUNK)], n_buf, sem).wait()

        def group_body(g, _):
            rows = g * L + lane

            def d_body(d, accs):
                acc_p, acc_n = accs
                col = jnp.full((L,), 0, jnp.int32) + d
                u = plsc.load_gather(u_buf, [rows, col])
                p = plsc.load_gather(p_buf, [rows, col])
                n = plsc.load_gather(n_buf, [rows, col])
                return (acc_p + u * p, acc_n + u * n)

            acc_p, acc_n = lax.fori_loop(
                0, D, d_body,
                (jnp.zeros((L,), jnp.float32), jnp.zeros((L,), jnp.float32)))
            o_p[pl.ds(coff + g * L, L)] = acc_p
            o_n[pl.ds(coff + g * L, L)] = acc_n
            return 0

        lax.fori_loop(0, CHUNK // L, group_body, 0)
        return 0

    lax.fori_loop(0, BPW // CHUNK, chunk_body, 0)

    pltpu.sync_copy(o_p, pos_out.at[pl.ds(base, BPW)])
    pltpu.sync_copy(o_n, neg_out.at[pl.ds(base, BPW)])


@functools.cache
def _build():
    mesh = plsc.VectorSubcoreMesh(core_axis_name="c", subcore_axis_name="s",
                                  num_cores=NC, num_subcores=NS)
    params = pltpu.CompilerParams(needs_layout_passes=False,
                                  use_tc_tiling_on_sc=True,
                                  disable_bounds_checks=True)
    sweep = pl.kernel(
        _sweep_sc,
        out_type=jax.ShapeDtypeStruct((3 * B, D), jnp.float32),
        mesh=mesh,
        scratch_types=[
            pltpu.VMEM((B,), jnp.int32),
            pltpu.VMEM((B,), jnp.int32),
            pltpu.VMEM((B,), jnp.int32),
            pltpu.VMEM((HMAX,), jnp.int32),
            pltpu.VMEM((HMAX,), jnp.int32),
            pltpu.VMEM((HMAX,), jnp.int32),
            pltpu.VMEM((HMAX,), jnp.int32),
            pltpu.VMEM((256,), jnp.int32),
            pltpu.VMEM((272,), jnp.int32),
            pltpu.VMEM((256,), jnp.int32),
            pltpu.VMEM((D, 128), jnp.float32),
            pltpu.VMEM((D, 128), jnp.float32),
            pltpu.VMEM((RB, D), jnp.float32),
            pltpu.VMEM((2 * D,), jnp.int32),
            pltpu.SemaphoreType.DMA,
            pltpu.SemaphoreType.DMA,
        ],
        compiler_params=params,
    )
    dots = pl.pallas_call(
        _dots_tc,
        grid=(B // _TCB,),
        in_specs=[
            pl.BlockSpec((_TCB, D), lambda i: (i, 0)),
            pl.BlockSpec((_TCB, D), lambda i: (B // _TCB + i, 0)),
            pl.BlockSpec((_TCB, D), lambda i: (2 * (B // _TCB) + i, 0)),
        ],
        out_specs=[
            pl.BlockSpec((_TCB,), lambda i: (i,)),
            pl.BlockSpec((_TCB,), lambda i: (i,)),
        ],
        out_shape=[
            jax.ShapeDtypeStruct((B,), jnp.float32),
            jax.ShapeDtypeStruct((B,), jnp.float32),
        ],
    )
    return sweep, dots


def kernel(users, pos, neg, E):
    sweep, dots = _build()
    staging = sweep(users.astype(jnp.int32), pos.astype(jnp.int32),
                    neg.astype(jnp.int32), E.T)
    out = dots(staging, staging, staging)
    return (out[0], out[1])
